# Initial kernel scaffold; baseline (speedup 1.0000x reference)
import jax
import jax.numpy as jnp
from jax.experimental import pallas as pl


def kernel(predictions, targets):
    raise NotImplementedError("write your pallas kernel here")


# speedup vs baseline: 77.0434x; 77.0434x over previous
import functools

import jax
import jax.numpy as jnp
from jax import lax
from jax.experimental import pallas as pl
from jax.experimental.pallas import tpu as pltpu
from jax.experimental.pallas import tpu_sc as plsc

_NUM_CLASSES = 19
_ALPHA = 0.25
_B = 16
_ROW = 512 * 512
_SPAN = _ROW // 2
_CHUNK = 32768
_NCHUNK = _SPAN // _CHUNK
_VECS2 = _CHUNK // 32


def _sc_presence_body(t_hbm, out_hbm, buf0, buf1, accv, sem0, sem1):
    c = lax.axis_index("c")
    s = lax.axis_index("s")
    base = (s * 2 + c) * _SPAN
    out_row = c * 16 + s
    bufs = [buf0, buf1]
    sems = [sem0, sem1]
    cps = [None, None]
    cps[0] = pltpu.async_copy(t_hbm.at[pl.ds(base, _CHUNK)], buf0, sem0)
    one = jnp.full((16,), 1, dtype=jnp.int32)
    acc0 = jnp.zeros((16,), jnp.int32)
    acc1 = jnp.zeros((16,), jnp.int32)
    for k in range(_NCHUNK):
        if k + 1 < _NCHUNK:
            nb = (k + 1) % 2
            cps[nb] = pltpu.async_copy(
                t_hbm.at[pl.ds(base + (k + 1) * _CHUNK, _CHUNK)], bufs[nb], sems[nb])
        cps[k % 2].wait()
        buf = bufs[k % 2]

        def _chunk_loop(i, carry, buf=buf):
            a0, a1 = carry
            j = pl.multiple_of(i * 32, 32)
            v0 = buf[pl.ds(j, 16)]
            v1 = buf[pl.ds(j + 16, 16)]
            return (a0 | (one << v0), a1 | (one << v1))

        acc0, acc1 = plsc.parallel_loop(
            0, _VECS2, unroll=4, carry=(acc0, acc1))(_chunk_loop)
    accv[...] = acc0 | acc1
    pltpu.sync_copy(accv, out_hbm.at[out_row])


_sc_presence = pl.kernel(
    _sc_presence_body,
    out_type=jax.ShapeDtypeStruct((2 * 16, 16), jnp.int32),
    mesh=plsc.VectorSubcoreMesh(core_axis_name="c", subcore_axis_name="s"),
    scratch_types=[
        pltpu.VMEM((_CHUNK,), jnp.int32),
        pltpu.VMEM((_CHUNK,), jnp.int32),
        pltpu.VMEM((16,), jnp.int32),
        pltpu.SemaphoreType.DMA,
        pltpu.SemaphoreType.DMA,
    ],
)


def _bce_body(pred_ref, masks_ref, out_ref):
    m = masks_ref[...]
    m = m[0:16, :] | m[16:32, :]
    m = m[:, 0:8] | m[:, 8:16]
    m = m[:, 0:4] | m[:, 4:8]
    m = m[:, 0:2] | m[:, 2:4]
    m = m[:, 0:1] | m[:, 1:2]
    cls = lax.broadcasted_iota(jnp.int32, (_B, _NUM_CLASSES), 1)
    enc = ((m >> cls) & 1).astype(jnp.float32)
    x = pred_ref[...]
    terms = jnp.maximum(x, 0.0) - x * enc + jnp.log1p(jnp.exp(-jnp.abs(x)))
    total = (_ALPHA / (_B * _NUM_CLASSES)) * jnp.sum(terms)
    out_ref[...] = jnp.reshape(total, (1, 1))


_bce_call = pl.pallas_call(
    _bce_body,
    out_shape=jax.ShapeDtypeStruct((1, 1), jnp.float32),
)


def kernel(predictions, targets):
    t_flat = targets.reshape(-1)
    masks = _sc_presence(t_flat)
    res = _bce_call(predictions, masks)
    return res[0, 0]


# speedup vs baseline: 137.8928x; 1.7898x over previous
import jax
import jax.numpy as jnp
from jax import lax
from jax.experimental import pallas as pl
from jax.experimental.pallas import tpu as pltpu
from jax.experimental.pallas import tpu_sc as plsc

_NUM_CLASSES = 19
_ALPHA = 0.25
_B = 16
_H = 512
_W = 512
_CROWS = 64
_NCHUNK = (_H // 2) // _CROWS


def _sc_presence_body(t_hbm, out_hbm, buf0, buf1, accv, sem0, sem1):
    c = lax.axis_index("c")
    s = lax.axis_index("s")
    half = c * (_H // 2)
    out_row = c * 16 + s
    bufs = [buf0, buf1]
    sems = [sem0, sem1]
    cps = [None, None]
    cps[0] = pltpu.async_copy(t_hbm.at[s, pl.ds(half, _CROWS), :], buf0, sem0)
    one = jnp.full((16,), 1, dtype=jnp.int32)
    acc0 = jnp.zeros((16,), jnp.int32)
    acc1 = jnp.zeros((16,), jnp.int32)
    for k in range(_NCHUNK):
        if k + 1 < _NCHUNK:
            nb = (k + 1) % 2
            cps[nb] = pltpu.async_copy(
                t_hbm.at[s, pl.ds(half + (k + 1) * _CROWS, _CROWS), :],
                bufs[nb], sems[nb])
        cps[k % 2].wait()
        buf = bufs[k % 2]

        def _chunk_loop(i, carry, buf=buf):
            a0, a1 = carry
            for jj in range(_W // 32):
                v0 = buf[i, pl.ds(jj * 32, 16)]
                v1 = buf[i, pl.ds(jj * 32 + 16, 16)]
                a0 = a0 | (one << v0)
                a1 = a1 | (one << v1)
            return (a0, a1)

        acc0, acc1 = plsc.parallel_loop(
            0, _CROWS, carry=(acc0, acc1))(_chunk_loop)
    accv[...] = acc0 | acc1
    pltpu.sync_copy(accv, out_hbm.at[out_row])


_sc_presence = pl.kernel(
    _sc_presence_body,
    out_type=jax.ShapeDtypeStruct((2 * 16, 16), jnp.int32),
    mesh=plsc.VectorSubcoreMesh(core_axis_name="c", subcore_axis_name="s"),
    compiler_params=pltpu.CompilerParams(use_tc_tiling_on_sc=True),
    scratch_types=[
        pltpu.VMEM((_CROWS, _W), jnp.int32),
        pltpu.VMEM((_CROWS, _W), jnp.int32),
        pltpu.VMEM((16,), jnp.int32),
        pltpu.SemaphoreType.DMA,
        pltpu.SemaphoreType.DMA,
    ],
)


def _bce_body(pred_ref, masks_ref, out_ref):
    m = masks_ref[...]
    m = m[0:16, :] | m[16:32, :]
    m = m[:, 0:8] | m[:, 8:16]
    m = m[:, 0:4] | m[:, 4:8]
    m = m[:, 0:2] | m[:, 2:4]
    m = m[:, 0:1] | m[:, 1:2]
    cls = lax.broadcasted_iota(jnp.int32, (_B, _NUM_CLASSES), 1)
    enc = ((m >> cls) & 1).astype(jnp.float32)
    x = pred_ref[...]
    terms = jnp.maximum(x, 0.0) - x * enc + jnp.log1p(jnp.exp(-jnp.abs(x)))
    total = (_ALPHA / (_B * _NUM_CLASSES)) * jnp.sum(terms)
    out_ref[...] = jnp.reshape(total, (1, 1))


_bce_call = pl.pallas_call(
    _bce_body,
    out_shape=jax.ShapeDtypeStruct((1, 1), jnp.float32),
)


def kernel(predictions, targets):
    masks = _sc_presence(targets)
    res = _bce_call(predictions, masks)
    return res[0, 0]
